# baseline (speedup 1.0000x reference)
import functools
import jax
import jax.numpy as jnp
from jax import lax
from jax.experimental import pallas as pl
from jax.experimental.pallas import tpu as pltpu
from jax.experimental.pallas import tpu_sc as plsc

N = 10000
E = 320000
D = 128
G = 64
DOUT = 40

NC = 2
NS = 16
NW = NC * NS
EPT = E // NW
EK = 80
NCHUNK = EPT // EK
RPT = N // NS

DEGW = 624


def _run_pipeline(process, fire, drain):
    fire(0, 0)
    fire(1, 1)
    process(0, 0)
    fire(2, 2)
    process(1, 1)
    drain(0)
    fire(3, 0)

    def body(t, _):
        k0 = 3 * t + 2
        process(k0, 2)
        drain(1)
        fire(k0 + 2, 1)
        process(k0 + 1, 0)
        drain(2)
        fire(k0 + 3, 2)
        process(k0 + 2, 1)
        drain(0)
        fire(k0 + 4, 0)
        return 0

    lax.fori_loop(0, (NCHUNK - 5) // 3, body, 0)
    process(NCHUNK - 3, 2)
    drain(1)
    fire(NCHUNK - 1, 1)
    process(NCHUNK - 2, 0)
    drain(2)
    process(NCHUNK - 1, 1)
    drain(0)
    drain(1)


def _deg_body(ei_hbm, ew_hbm, deg_hbm, cidx_v, ewc_v, zb_v, deg_sh,
              sems, ssems):
    c = lax.axis_index("c")
    s = lax.axis_index("s")
    wid = c * NS + s

    z = jnp.zeros((16,), jnp.float32)

    def zb(i, _):
        zb_v[pl.ds(i * 16, 16)] = z
        return 0

    lax.fori_loop(0, (DEGW + 16) // 16, zb, 0)
    pltpu.sync_copy(zb_v.at[pl.ds(0, DEGW)], deg_sh.at[pl.ds(s * DEGW, DEGW)])

    @pl.when(s == NS - 1)
    def _():
        pltpu.sync_copy(zb_v.at[pl.ds(0, 16)], deg_sh.at[pl.ds(NS * DEGW, 16)])

    plsc.subcore_barrier()

    def fire(k, b):
        cb = pl.multiple_of(E + wid * EPT + k * EK, 8)
        pltpu.async_copy(ei_hbm.at[pl.ds(cb, EK)], cidx_v.at[b], sems[b])
        pltpu.async_copy(ew_hbm.at[pl.ds(cb - E, EK)], ewc_v.at[b], sems[b])

    def process(k, b):
        pltpu.make_async_copy(ei_hbm.at[pl.ds(0, EK)], cidx_v.at[b],
                              sems[b]).wait()
        pltpu.make_async_copy(ew_hbm.at[pl.ds(0, EK)], ewc_v.at[b],
                              sems[b]).wait()
        pltpu.async_copy(ewc_v.at[b], deg_sh.at[cidx_v.at[b]], ssems[b],
                         add=True)

    def drain(b):
        pltpu.make_async_copy(ewc_v.at[b], deg_sh.at[cidx_v.at[b]],
                              ssems[b]).wait()

    _run_pipeline(process, fire, drain)
    plsc.subcore_barrier()

    obase = pl.multiple_of(c * N + s * DEGW, 8)
    pltpu.sync_copy(deg_sh.at[pl.ds(s * DEGW, DEGW)], zb_v.at[pl.ds(0, DEGW)])
    pltpu.sync_copy(zb_v.at[pl.ds(0, DEGW)], deg_hbm.at[pl.ds(obase, DEGW)])

    @pl.when(s == NS - 1)
    def _():
        tbase = pl.multiple_of(c * N + NS * DEGW, 8)
        pltpu.sync_copy(deg_sh.at[pl.ds(NS * DEGW, 16)],
                        zb_v.at[pl.ds(DEGW, 16)])
        pltpu.sync_copy(zb_v.at[pl.ds(DEGW, 16)], deg_hbm.at[pl.ds(tbase, 16)])


@functools.partial(
    pl.kernel,
    out_type=jax.ShapeDtypeStruct((NC * N,), jnp.float32),
    mesh=plsc.VectorSubcoreMesh(core_axis_name="c", subcore_axis_name="s"),
    scratch_types=[
        pltpu.VMEM((3, EK), jnp.int32),
        pltpu.VMEM((3, EK), jnp.float32),
        pltpu.VMEM((DEGW + 16,), jnp.float32),
        pltpu.VMEM_SHARED((N,), jnp.float32),
        pltpu.SemaphoreType.DMA,
        pltpu.SemaphoreType.DMA,
        pltpu.SemaphoreType.DMA,
        pltpu.SemaphoreType.DMA,
        pltpu.SemaphoreType.DMA,
        pltpu.SemaphoreType.DMA,
    ],
)
def _deg_sc(ei_hbm, ew_hbm, deg_hbm, cidx_v, ewc_v, zb_v, deg_sh,
            g0, g1, g2, s0, s1, s2):
    _deg_body(ei_hbm, ew_hbm, deg_hbm, cidx_v, ewc_v, zb_v, deg_sh,
              (g0, g1, g2), (s0, s1, s2))


def _edge_body(xw_hbm, ei_hbm, ew_hbm, acc_hbm,
               ridx_v, cidx_v, ewc_v, rows_v, acc_sh, sems, ssems):
    c = lax.axis_index("c")
    s = lax.axis_index("s")
    wid = c * NS + s

    z = jnp.zeros((16,), jnp.float32)

    def zrow(i, _):
        for j in range(D // 16):
            rows_v[0, i, pl.ds(j * 16, 16)] = z
        return 0

    lax.fori_loop(0, EK, zrow, 0)
    for k in range(RPT // EK):
        pltpu.sync_copy(rows_v.at[0],
                        acc_sh.at[pl.ds(s * RPT + k * EK, EK)])
    rem = RPT % EK
    if rem:
        pltpu.sync_copy(rows_v.at[0, pl.ds(0, rem)],
                        acc_sh.at[pl.ds(s * RPT + (RPT // EK) * EK, rem)])
    plsc.subcore_barrier()

    tbase = pl.multiple_of(wid * EPT, 8)
    pltpu.sync_copy(ei_hbm.at[pl.ds(tbase, EPT)], ridx_v)

    def fire(k, b):
        cb = pl.multiple_of(E + wid * EPT + k * EK, 8)
        pltpu.async_copy(ei_hbm.at[pl.ds(cb, EK)], cidx_v.at[b], sems[b])
        pltpu.async_copy(ew_hbm.at[pl.ds(cb - E, EK)], ewc_v.at[b], sems[b])
        pltpu.async_copy(xw_hbm.at[ridx_v.at[pl.ds(k * EK, EK)]],
                         rows_v.at[b], sems[b])

    def process(k, b):
        pltpu.make_async_copy(ei_hbm.at[pl.ds(0, EK)], cidx_v.at[b],
                              sems[b]).wait()
        pltpu.make_async_copy(ew_hbm.at[pl.ds(0, EK)], ewc_v.at[b],
                              sems[b]).wait()
        pltpu.make_async_copy(xw_hbm.at[ridx_v.at[pl.ds(0, EK)]],
                              rows_v.at[b], sems[b]).wait()

        def scale_group(g, _):
            wv = ewc_v[b, pl.ds(g * 16, 16)]
            for l in range(16):
                w = wv[l]
                r = g * 16 + l
                for j in range(D // 16):
                    rows_v[b, r, pl.ds(j * 16, 16)] = (
                        rows_v[b, r, pl.ds(j * 16, 16)] * w)
            return 0

        lax.fori_loop(0, EK // 16, scale_group, 0)
        pltpu.async_copy(rows_v.at[b], acc_sh.at[cidx_v.at[b]], ssems[b],
                         add=True)

    def drain(b):
        pltpu.make_async_copy(rows_v.at[b], acc_sh.at[cidx_v.at[b]],
                              ssems[b]).wait()

    _run_pipeline(process, fire, drain)
    plsc.subcore_barrier()

    rbase = pl.multiple_of(s * DEGW, 8)
    pltpu.sync_copy(acc_sh.at[pl.ds(rbase, DEGW)],
                    acc_hbm.at[c, pl.ds(rbase, DEGW)])

    @pl.when(s == NS - 1)
    def _():
        pltpu.sync_copy(acc_sh.at[pl.ds(NS * DEGW, 16)],
                        acc_hbm.at[c, pl.ds(NS * DEGW, 16)])


@functools.partial(
    pl.kernel,
    out_type=jax.ShapeDtypeStruct((NC, N, D), jnp.float32),
    mesh=plsc.VectorSubcoreMesh(core_axis_name="c", subcore_axis_name="s"),
    scratch_types=[
        pltpu.VMEM((EPT,), jnp.int32),
        pltpu.VMEM((3, EK), jnp.int32),
        pltpu.VMEM((3, EK), jnp.float32),
        pltpu.VMEM((3, EK, D), jnp.float32),
        pltpu.VMEM_SHARED((N, D), jnp.float32),
        pltpu.SemaphoreType.DMA,
        pltpu.SemaphoreType.DMA,
        pltpu.SemaphoreType.DMA,
        pltpu.SemaphoreType.DMA,
        pltpu.SemaphoreType.DMA,
        pltpu.SemaphoreType.DMA,
    ],
)
def _edge_sc(xw_hbm, ei_hbm, ew_hbm, acc_hbm,
             ridx_v, cidx_v, ewc_v, rows_v, acc_sh,
             g0, g1, g2, s0, s1, s2):
    _edge_body(xw_hbm, ei_hbm, ew_hbm, acc_hbm,
               ridx_v, cidx_v, ewc_v, rows_v, acc_sh,
               (g0, g1, g2), (s0, s1, s2))


def _mmraw_body(x_ref, w_ref, xw_ref):
    xw_ref[...] = jnp.dot(x_ref[...], w_ref[...],
                          preferred_element_type=jnp.float32)


def _mmraw(x, W):
    return pl.pallas_call(
        _mmraw_body,
        out_shape=jax.ShapeDtypeStruct((N, D), jnp.float32),
    )(x, W)


def _scale1_body(xwr_ref, deg_ref, xw_ref, dis_ref):
    deg = deg_ref[0] + deg_ref[1] + 1.0
    dis = lax.rsqrt(deg)
    dis_ref[...] = dis
    xw_ref[...] = xwr_ref[...] * dis


def _scale1(xwr, deg2):
    return pl.pallas_call(
        _scale1_body,
        out_shape=(jax.ShapeDtypeStruct((N, D), jnp.float32),
                   jax.ShapeDtypeStruct((N, 1), jnp.float32)),
    )(xwr, deg2)


def _finish_mm_body(acc_ref, xwp_ref, dis_ref, b_ref, w_ref, out_ref):
    dis = dis_ref[...]
    h = dis * (acc_ref[0] + acc_ref[1] + xwp_ref[...]) + b_ref[...]
    h = jnp.maximum(h, 0.0)
    out_ref[...] = jnp.dot(h, w_ref[...],
                           preferred_element_type=jnp.float32) * dis


def _finish_mm(acc, xwp, dis, b, W):
    return pl.pallas_call(
        _finish_mm_body,
        out_shape=jax.ShapeDtypeStruct((N, D), jnp.float32),
    )(acc, xwp, dis, b, W)


def _pool_head_body(acc_ref, xwp_ref, dis_ref, b_ref, batch_ref,
                    wout_ref, bout_ref, out_ref):
    h = dis_ref[...] * (acc_ref[0] + acc_ref[1] + xwp_ref[...]) + b_ref[...]
    h = jnp.maximum(h, 0.0)
    gids = lax.broadcasted_iota(jnp.int32, (1, G), 1)
    onehot = (batch_ref[...] == gids).astype(jnp.float32)
    ssum = lax.dot_general(onehot, h, (((0,), (0,)), ((), ())),
                           preferred_element_type=jnp.float32)
    ones = jnp.ones((N, 1), jnp.float32)
    cnt = lax.dot_general(onehot, ones, (((0,), (0,)), ((), ())),
                          preferred_element_type=jnp.float32)
    pooled = ssum / jnp.maximum(cnt, 1.0)
    logits = jnp.dot(pooled, wout_ref[...],
                     preferred_element_type=jnp.float32) + bout_ref[...]
    m = jnp.max(logits, axis=1, keepdims=True)
    lse = m + jnp.log(jnp.sum(jnp.exp(logits - m), axis=1, keepdims=True))
    out_ref[...] = logits - lse


def _pool_head(acc, xwp, dis, b, batch, Wout, bout):
    return pl.pallas_call(
        _pool_head_body,
        out_shape=jax.ShapeDtypeStruct((G, DOUT), jnp.float32),
    )(acc, xwp, dis, b, batch, Wout, bout)


def kernel(x, edge_idx, edge_weight, batch, W1, b1, W2, b2, W3, b3,
           Wout, bout):
    ei = edge_idx.reshape(2 * E)
    xwr = _mmraw(x, W1)
    deg2 = _deg_sc(ei, edge_weight)
    xw1, dis = _scale1(xwr, deg2.reshape(NC, N, 1))
    acc = _edge_sc(xw1, ei, edge_weight)
    xw2 = _finish_mm(acc, xw1, dis, b1.reshape(1, D), W2)
    acc = _edge_sc(xw2, ei, edge_weight)
    xw3 = _finish_mm(acc, xw2, dis, b2.reshape(1, D), W3)
    acc = _edge_sc(xw3, ei, edge_weight)
    return _pool_head(acc, xw3, dis, b3.reshape(1, D),
                      batch.reshape(N, 1), Wout, bout.reshape(1, DOUT))


# speedup vs baseline: 1.0111x; 1.0111x over previous
import functools
import jax
import jax.numpy as jnp
from jax import lax
from jax.experimental import pallas as pl
from jax.experimental.pallas import tpu as pltpu
from jax.experimental.pallas import tpu_sc as plsc

N = 10000
E = 320000
D = 128
G = 64
DOUT = 40

NC = 2
NS = 16
NW = NC * NS
EPT = E // NW
EK = 80
NCHUNK = EPT // EK
RPT = N // NS

DEGW = 624


def _run_pipeline(process, fire, drain, after_prologue=None):
    fire(0, 0)
    fire(1, 1)
    if after_prologue is not None:
        after_prologue()
    process(0, 0)
    fire(2, 2)
    process(1, 1)
    drain(0)
    fire(3, 0)

    def body(t, _):
        k0 = 3 * t + 2
        process(k0, 2)
        drain(1)
        fire(k0 + 2, 1)
        process(k0 + 1, 0)
        drain(2)
        fire(k0 + 3, 2)
        process(k0 + 2, 1)
        drain(0)
        fire(k0 + 4, 0)
        return 0

    lax.fori_loop(0, (NCHUNK - 5) // 3, body, 0)
    process(NCHUNK - 3, 2)
    drain(1)
    fire(NCHUNK - 1, 1)
    process(NCHUNK - 2, 0)
    drain(2)
    process(NCHUNK - 1, 1)
    drain(0)
    drain(1)


def _deg_body(ei_hbm, ew_hbm, deg_hbm, cidx_v, ewc_v, zb_v, deg_sh,
              sems, ssems):
    c = lax.axis_index("c")
    s = lax.axis_index("s")
    wid = c * NS + s

    z = jnp.zeros((16,), jnp.float32)

    def zb(i, _):
        zb_v[pl.ds(i * 16, 16)] = z
        return 0

    lax.fori_loop(0, (DEGW + 16) // 16, zb, 0)
    pltpu.sync_copy(zb_v.at[pl.ds(0, DEGW)], deg_sh.at[pl.ds(s * DEGW, DEGW)])

    @pl.when(s == NS - 1)
    def _():
        pltpu.sync_copy(zb_v.at[pl.ds(0, 16)], deg_sh.at[pl.ds(NS * DEGW, 16)])

    plsc.subcore_barrier()

    def fire(k, b):
        cb = pl.multiple_of(E + wid * EPT + k * EK, 8)
        pltpu.async_copy(ei_hbm.at[pl.ds(cb, EK)], cidx_v.at[b], sems[b])
        pltpu.async_copy(ew_hbm.at[pl.ds(cb - E, EK)], ewc_v.at[b], sems[b])

    def process(k, b):
        pltpu.make_async_copy(ei_hbm.at[pl.ds(0, EK)], cidx_v.at[b],
                              sems[b]).wait()
        pltpu.make_async_copy(ew_hbm.at[pl.ds(0, EK)], ewc_v.at[b],
                              sems[b]).wait()
        pltpu.async_copy(ewc_v.at[b], deg_sh.at[cidx_v.at[b]], ssems[b],
                         add=True)

    def drain(b):
        pltpu.make_async_copy(ewc_v.at[b], deg_sh.at[cidx_v.at[b]],
                              ssems[b]).wait()

    _run_pipeline(process, fire, drain)
    plsc.subcore_barrier()

    obase = pl.multiple_of(c * N + s * DEGW, 8)
    pltpu.sync_copy(deg_sh.at[pl.ds(s * DEGW, DEGW)], zb_v.at[pl.ds(0, DEGW)])
    pltpu.sync_copy(zb_v.at[pl.ds(0, DEGW)], deg_hbm.at[pl.ds(obase, DEGW)])

    @pl.when(s == NS - 1)
    def _():
        tbase = pl.multiple_of(c * N + NS * DEGW, 8)
        pltpu.sync_copy(deg_sh.at[pl.ds(NS * DEGW, 16)],
                        zb_v.at[pl.ds(DEGW, 16)])
        pltpu.sync_copy(zb_v.at[pl.ds(DEGW, 16)], deg_hbm.at[pl.ds(tbase, 16)])


@functools.partial(
    pl.kernel,
    out_type=jax.ShapeDtypeStruct((NC * N,), jnp.float32),
    mesh=plsc.VectorSubcoreMesh(core_axis_name="c", subcore_axis_name="s"),
    scratch_types=[
        pltpu.VMEM((3, EK), jnp.int32),
        pltpu.VMEM((3, EK), jnp.float32),
        pltpu.VMEM((DEGW + 16,), jnp.float32),
        pltpu.VMEM_SHARED((N,), jnp.float32),
        pltpu.SemaphoreType.DMA,
        pltpu.SemaphoreType.DMA,
        pltpu.SemaphoreType.DMA,
        pltpu.SemaphoreType.DMA,
        pltpu.SemaphoreType.DMA,
        pltpu.SemaphoreType.DMA,
    ],
)
def _deg_sc(ei_hbm, ew_hbm, deg_hbm, cidx_v, ewc_v, zb_v, deg_sh,
            g0, g1, g2, s0, s1, s2):
    _deg_body(ei_hbm, ew_hbm, deg_hbm, cidx_v, ewc_v, zb_v, deg_sh,
              (g0, g1, g2), (s0, s1, s2))


def _edge_body(xw_hbm, ei_hbm, ew_hbm, acc_hbm,
               ridx_v, cidx_v, ewc_v, rows_v, acc_sh, sems, ssems):
    c = lax.axis_index("c")
    s = lax.axis_index("s")
    wid = c * NS + s

    tbase = pl.multiple_of(wid * EPT, 8)
    pltpu.sync_copy(ei_hbm.at[pl.ds(tbase, EPT)], ridx_v)

    def fire(k, b):
        cb = pl.multiple_of(E + wid * EPT + k * EK, 8)
        pltpu.async_copy(ei_hbm.at[pl.ds(cb, EK)], cidx_v.at[b], sems[b])
        pltpu.async_copy(ew_hbm.at[pl.ds(cb - E, EK)], ewc_v.at[b], sems[b])
        pltpu.async_copy(xw_hbm.at[ridx_v.at[pl.ds(k * EK, EK)]],
                         rows_v.at[b], sems[b])

    def process(k, b):
        pltpu.make_async_copy(ei_hbm.at[pl.ds(0, EK)], cidx_v.at[b],
                              sems[b]).wait()
        pltpu.make_async_copy(ew_hbm.at[pl.ds(0, EK)], ewc_v.at[b],
                              sems[b]).wait()
        pltpu.make_async_copy(xw_hbm.at[ridx_v.at[pl.ds(0, EK)]],
                              rows_v.at[b], sems[b]).wait()

        def scale_group(g, _):
            wv = ewc_v[b, pl.ds(g * 16, 16)]
            for l in range(16):
                w = wv[l]
                r = g * 16 + l
                for j in range(D // 16):
                    rows_v[b, r, pl.ds(j * 16, 16)] = (
                        rows_v[b, r, pl.ds(j * 16, 16)] * w)
            return 0

        lax.fori_loop(0, EK // 16, scale_group, 0)
        pltpu.async_copy(rows_v.at[b], acc_sh.at[cidx_v.at[b]], ssems[b],
                         add=True)

    def drain(b):
        pltpu.make_async_copy(rows_v.at[b], acc_sh.at[cidx_v.at[b]],
                              ssems[b]).wait()

    def init_acc():
        z = jnp.zeros((16,), jnp.float32)

        def zrow(i, _):
            for j in range(D // 16):
                rows_v[2, i, pl.ds(j * 16, 16)] = z
            return 0

        lax.fori_loop(0, EK, zrow, 0)
        for k in range(RPT // EK):
            pltpu.sync_copy(rows_v.at[2],
                            acc_sh.at[pl.ds(s * RPT + k * EK, EK)])
        rem = RPT % EK
        if rem:
            pltpu.sync_copy(rows_v.at[2, pl.ds(0, rem)],
                            acc_sh.at[pl.ds(s * RPT + (RPT // EK) * EK, rem)])
        plsc.subcore_barrier()

    _run_pipeline(process, fire, drain, init_acc)
    plsc.subcore_barrier()

    rbase = pl.multiple_of(s * DEGW, 8)
    pltpu.sync_copy(acc_sh.at[pl.ds(rbase, DEGW)],
                    acc_hbm.at[c, pl.ds(rbase, DEGW)])

    @pl.when(s == NS - 1)
    def _():
        pltpu.sync_copy(acc_sh.at[pl.ds(NS * DEGW, 16)],
                        acc_hbm.at[c, pl.ds(NS * DEGW, 16)])


@functools.partial(
    pl.kernel,
    out_type=jax.ShapeDtypeStruct((NC, N, D), jnp.float32),
    mesh=plsc.VectorSubcoreMesh(core_axis_name="c", subcore_axis_name="s"),
    scratch_types=[
        pltpu.VMEM((EPT,), jnp.int32),
        pltpu.VMEM((3, EK), jnp.int32),
        pltpu.VMEM((3, EK), jnp.float32),
        pltpu.VMEM((3, EK, D), jnp.float32),
        pltpu.VMEM_SHARED((N, D), jnp.float32),
        pltpu.SemaphoreType.DMA,
        pltpu.SemaphoreType.DMA,
        pltpu.SemaphoreType.DMA,
        pltpu.SemaphoreType.DMA,
        pltpu.SemaphoreType.DMA,
        pltpu.SemaphoreType.DMA,
    ],
)
def _edge_sc(xw_hbm, ei_hbm, ew_hbm, acc_hbm,
             ridx_v, cidx_v, ewc_v, rows_v, acc_sh,
             g0, g1, g2, s0, s1, s2):
    _edge_body(xw_hbm, ei_hbm, ew_hbm, acc_hbm,
               ridx_v, cidx_v, ewc_v, rows_v, acc_sh,
               (g0, g1, g2), (s0, s1, s2))


def _mm1_body(x_ref, w_ref, deg_ref, xw_ref, dis_ref):
    deg = deg_ref[0] + deg_ref[1] + 1.0
    dis = lax.rsqrt(deg)
    dis_ref[...] = dis
    xw = jnp.dot(x_ref[...], w_ref[...], preferred_element_type=jnp.float32)
    xw_ref[...] = xw * dis


def _mm1(x, W, deg2):
    return pl.pallas_call(
        _mm1_body,
        out_shape=(jax.ShapeDtypeStruct((N, D), jnp.float32),
                   jax.ShapeDtypeStruct((N, 1), jnp.float32)),
    )(x, W, deg2)


def _finish_mm_body(acc_ref, xwp_ref, dis_ref, b_ref, w_ref, out_ref):
    dis = dis_ref[...]
    h = dis * (acc_ref[0] + acc_ref[1] + xwp_ref[...]) + b_ref[...]
    h = jnp.maximum(h, 0.0)
    out_ref[...] = jnp.dot(h, w_ref[...],
                           preferred_element_type=jnp.float32) * dis


def _finish_mm(acc, xwp, dis, b, W):
    return pl.pallas_call(
        _finish_mm_body,
        out_shape=jax.ShapeDtypeStruct((N, D), jnp.float32),
    )(acc, xwp, dis, b, W)


def _pool_head_body(acc_ref, xwp_ref, dis_ref, b_ref, batch_ref,
                    wout_ref, bout_ref, out_ref):
    h = dis_ref[...] * (acc_ref[0] + acc_ref[1] + xwp_ref[...]) + b_ref[...]
    h = jnp.maximum(h, 0.0)
    gids = lax.broadcasted_iota(jnp.int32, (1, G), 1)
    onehot = (batch_ref[...] == gids).astype(jnp.float32)
    ssum = lax.dot_general(onehot, h, (((0,), (0,)), ((), ())),
                           preferred_element_type=jnp.float32)
    ones = jnp.ones((N, 1), jnp.float32)
    cnt = lax.dot_general(onehot, ones, (((0,), (0,)), ((), ())),
                          preferred_element_type=jnp.float32)
    pooled = ssum / jnp.maximum(cnt, 1.0)
    logits = jnp.dot(pooled, wout_ref[...],
                     preferred_element_type=jnp.float32) + bout_ref[...]
    m = jnp.max(logits, axis=1, keepdims=True)
    lse = m + jnp.log(jnp.sum(jnp.exp(logits - m), axis=1, keepdims=True))
    out_ref[...] = logits - lse


def _pool_head(acc, xwp, dis, b, batch, Wout, bout):
    return pl.pallas_call(
        _pool_head_body,
        out_shape=jax.ShapeDtypeStruct((G, DOUT), jnp.float32),
    )(acc, xwp, dis, b, batch, Wout, bout)


def kernel(x, edge_idx, edge_weight, batch, W1, b1, W2, b2, W3, b3,
           Wout, bout):
    ei = edge_idx.reshape(2 * E)
    deg2 = _deg_sc(ei, edge_weight)
    xw1, dis = _mm1(x, W1, deg2.reshape(NC, N, 1))
    acc = _edge_sc(xw1, ei, edge_weight)
    xw2 = _finish_mm(acc, xw1, dis, b1.reshape(1, D), W2)
    acc = _edge_sc(xw2, ei, edge_weight)
    xw3 = _finish_mm(acc, xw2, dis, b2.reshape(1, D), W3)
    acc = _edge_sc(xw3, ei, edge_weight)
    return _pool_head(acc, xw3, dis, b3.reshape(1, D),
                      batch.reshape(N, 1), Wout, bout.reshape(1, DOUT))
